# baseline (speedup 1.0000x reference)
import functools

import jax
import jax.numpy as jnp
from jax import lax
from jax.experimental import pallas as pl
from jax.experimental.pallas import tpu as pltpu
from jax.experimental.pallas import tpu_sc as plsc

EMBED = 64
BATCH = 16384
NC, NS, L = 2, 16, 16
NW = NC * NS
BPW = BATCH // NW
NGROUPS = BPW // L


def _sc_dots(center_ids, context_ids, W_center, W_context):
    mesh = plsc.VectorSubcoreMesh(
        core_axis_name="c", subcore_axis_name="s",
        num_cores=NC, num_subcores=NS)

    @functools.partial(
        pl.kernel,
        out_type=jax.ShapeDtypeStruct((BATCH,), jnp.float32),
        mesh=mesh,
        compiler_params=pltpu.CompilerParams(
            needs_layout_passes=False, use_tc_tiling_on_sc=False),
        scratch_types=[
            pltpu.VMEM((BPW,), jnp.int32),
            pltpu.VMEM((BPW,), jnp.int32),
            pltpu.VMEM((BPW, EMBED), jnp.float32),
            pltpu.VMEM((BPW, EMBED), jnp.float32),
            pltpu.VMEM((BPW,), jnp.float32),
            pltpu.SemaphoreType.DMA,
            pltpu.SemaphoreType.DMA,
        ],
    )
    def k(cid_hbm, oid_hbm, wc_hbm, wo_hbm, out_hbm,
          cidx_v, oidx_v, crows_v, orows_v, dots_v, sem_c, sem_o):
        wid = lax.axis_index("s") * NC + lax.axis_index("c")
        base = wid * BPW
        pltpu.sync_copy(cid_hbm.at[pl.ds(base, BPW)], cidx_v)
        pltpu.sync_copy(oid_hbm.at[pl.ds(base, BPW)], oidx_v)
        cp_c = pltpu.async_copy(wc_hbm.at[cidx_v], crows_v, sem_c)
        cp_o = pltpu.async_copy(wo_hbm.at[oidx_v], orows_v, sem_o)
        cp_c.wait()
        cp_o.wait()

        lanes = lax.iota(jnp.int32, L)

        def group_body(g, _):
            acc = jnp.zeros((L,), jnp.float32)
            for r in range(L):
                i = g * L + r
                rc = crows_v.at[i]
                ro = orows_v.at[i]
                part = rc[pl.ds(0, L)] * ro[pl.ds(0, L)]
                for k in range(1, EMBED // L):
                    part += rc[pl.ds(k * L, L)] * ro[pl.ds(k * L, L)]
                acc = jnp.where(lanes == r, jnp.sum(part), acc)
            dots_v[pl.ds(g * L, L)] = acc
            return 0

        lax.fori_loop(0, NGROUPS, group_body, 0)
        pltpu.sync_copy(dots_v, out_hbm.at[pl.ds(base, BPW)])

    return k(center_ids, context_ids, W_center, W_context)


def _tc_loss(dots):
    x = dots.reshape(BATCH // 128, 128)

    def body(x_ref, o_ref):
        v = x_ref[...]
        ls = jnp.minimum(v, 0.0) - jnp.log1p(jnp.exp(-jnp.abs(v)))
        o_ref[0, 0] = -jnp.sum(ls) / BATCH

    out = pl.pallas_call(
        body,
        out_shape=jax.ShapeDtypeStruct((1, 1), jnp.float32),
        out_specs=pl.BlockSpec(memory_space=pltpu.SMEM),
    )(x)
    return out[0, 0]


def kernel(center_ids, context_ids, W_center, W_context):
    dots = _sc_dots(center_ids.astype(jnp.int32),
                    context_ids.astype(jnp.int32),
                    W_center, W_context)
    return _tc_loss(dots)


# speedup vs baseline: 1.5730x; 1.5730x over previous
import functools

import jax
import jax.numpy as jnp
from jax import lax
from jax.experimental import pallas as pl
from jax.experimental.pallas import tpu as pltpu
from jax.experimental.pallas import tpu_sc as plsc

VOCAB = 1000000
EMBED = 64
BATCH = 16384
NC, NS, L = 2, 16, 16
NW = NC * NS
BPW = BATCH // NW
HALFB = BPW // 2
CH = 2048
NG = 245
OUTR = NG * CH


def _tc_pack(wt):
    def body(x0_ref, x1_ref, o_ref):
        o_ref[:, 0:EMBED] = x0_ref[...].T
        o_ref[:, EMBED:128] = x1_ref[...].T

    return pl.pallas_call(
        body,
        grid=(NG,),
        in_specs=[
            pl.BlockSpec((EMBED, CH), lambda g: (0, 2 * g)),
            pl.BlockSpec((EMBED, CH), lambda g: (0, jnp.minimum(2 * g + 1, 488))),
        ],
        out_specs=pl.BlockSpec((CH, 128), lambda g: (g, 0)),
        out_shape=jax.ShapeDtypeStruct((OUTR, 128), jnp.float32),
    )(wt, wt)


def _sc_dots(center_ids, context_ids, wc_packed, wo_packed):
    mesh = plsc.VectorSubcoreMesh(
        core_axis_name="c", subcore_axis_name="s",
        num_cores=NC, num_subcores=NS)

    @functools.partial(
        pl.kernel,
        out_type=jax.ShapeDtypeStruct((BATCH,), jnp.float32),
        mesh=mesh,
        compiler_params=pltpu.CompilerParams(needs_layout_passes=False),
        scratch_types=[
            pltpu.VMEM((BPW,), jnp.int32),
            pltpu.VMEM((BPW,), jnp.int32),
            pltpu.VMEM((BPW,), jnp.int32),
            pltpu.VMEM((BPW,), jnp.int32),
            pltpu.VMEM((BPW,), jnp.int32),
            pltpu.VMEM((BPW,), jnp.int32),
            pltpu.VMEM((HALFB, 128), jnp.float32),
            pltpu.VMEM((HALFB, 128), jnp.float32),
            pltpu.VMEM((BPW,), jnp.float32),
            pltpu.SemaphoreType.DMA,
            pltpu.SemaphoreType.DMA,
        ],
    )
    def k(cid_hbm, oid_hbm, wc_hbm, wo_hbm, out_hbm,
          cidx_v, oidx_v, crow_v, orow_v, chalf_v, ohalf_v,
          cbuf, obuf, dots_v, sem_c, sem_o):
        wid = lax.axis_index("s") * NC + lax.axis_index("c")
        base = wid * BPW
        pltpu.sync_copy(cid_hbm.at[pl.ds(base, BPW)], cidx_v)
        pltpu.sync_copy(oid_hbm.at[pl.ds(base, BPW)], oidx_v)

        def idx_body(t, _):
            s = t * L
            for src, row, half in ((cidx_v, crow_v, chalf_v),
                                   (oidx_v, orow_v, ohalf_v)):
                iv = src[pl.ds(s, L)]
                row[pl.ds(s, L)] = (iv & (CH - 1)) + ((iv >> 12) << 11)
                half[pl.ds(s, L)] = ((iv >> 11) & 1) * EMBED
            return 0

        lax.fori_loop(0, BPW // L, idx_body, 0)

        lanes = lax.iota(jnp.int32, L)
        for p in range(2):
            off = p * HALFB
            cp_c = pltpu.async_copy(
                wc_hbm.at[crow_v.at[pl.ds(off, HALFB)]], cbuf, sem_c)
            cp_o = pltpu.async_copy(
                wo_hbm.at[orow_v.at[pl.ds(off, HALFB)]], obuf, sem_o)
            cp_c.wait()
            cp_o.wait()

            def group_body(g, _, off=off):
                rows = g * L + lanes
                hc = chalf_v[pl.ds(off + g * L, L)]
                ho = ohalf_v[pl.ds(off + g * L, L)]

                def d_body(d, acc):
                    cv = plsc.load_gather(cbuf, [rows, hc + d])
                    ov = plsc.load_gather(obuf, [rows, ho + d])
                    return acc + cv * ov

                acc = lax.fori_loop(0, EMBED, d_body,
                                    jnp.zeros((L,), jnp.float32))
                dots_v[pl.ds(off + g * L, L)] = acc
                return 0

            lax.fori_loop(0, HALFB // L, group_body, 0)

        pltpu.sync_copy(dots_v, out_hbm.at[pl.ds(base, BPW)])

    return k(center_ids, context_ids, wc_packed, wo_packed)


def _tc_loss(dots):
    x = dots.reshape(BATCH // 128, 128)

    def body(x_ref, o_ref):
        v = x_ref[...]
        ls = jnp.minimum(v, 0.0) - jnp.log1p(jnp.exp(-jnp.abs(v)))
        o_ref[0, 0] = -jnp.sum(ls) / BATCH

    out = pl.pallas_call(
        body,
        out_shape=jax.ShapeDtypeStruct((1, 1), jnp.float32),
        out_specs=pl.BlockSpec(memory_space=pltpu.SMEM),
    )(x)
    return out[0, 0]


def kernel(center_ids, context_ids, W_center, W_context):
    wc_packed = _tc_pack(W_center.T)
    wo_packed = _tc_pack(W_context.T)
    dots = _sc_dots(center_ids.astype(jnp.int32),
                    context_ids.astype(jnp.int32),
                    wc_packed, wo_packed)
    return _tc_loss(dots)
